# baseline (speedup 1.0000x reference)
import jax
import jax.numpy as jnp
from jax.experimental import pallas as pl
from jax.experimental.pallas import tpu as pltpu

_BF = 1000
_BI = 200


def _support_body(f_ref, w_ref, o_ref):
    o_ref[...] = jnp.dot(
        f_ref[...].astype(jnp.bfloat16),
        w_ref[...],
        preferred_element_type=jnp.float32,
    ).astype(jnp.bfloat16)


def _spmm_body(adj_ref, s_ref, o_ref):
    acc = jnp.dot(
        adj_ref[...].astype(jnp.bfloat16),
        s_ref[...],
        preferred_element_type=jnp.float32,
    )
    o_ref[...] = jnp.maximum(acc, 0.0)


def kernel(features, adj, weight):
    n, d_in = features.shape
    d_out = weight.shape[1]

    support = pl.pallas_call(
        _support_body,
        grid=(n // _BF,),
        in_specs=[
            pl.BlockSpec((_BF, d_in), lambda i: (i, 0)),
            pl.BlockSpec((d_in, d_out), lambda i: (0, 0)),
        ],
        out_specs=pl.BlockSpec((_BF, d_out), lambda i: (i, 0)),
        out_shape=jax.ShapeDtypeStruct((n, d_out), jnp.bfloat16),
        compiler_params=pltpu.CompilerParams(
            dimension_semantics=("parallel",),
        ),
    )(features, weight.astype(jnp.bfloat16))

    out = pl.pallas_call(
        _spmm_body,
        grid=(n // _BI,),
        in_specs=[
            pl.BlockSpec((_BI, n), lambda i: (i, 0)),
            pl.BlockSpec((n, d_out), lambda i: (0, 0)),
        ],
        out_specs=pl.BlockSpec((_BI, d_out), lambda i: (i, 0)),
        out_shape=jax.ShapeDtypeStruct((n, d_out), jnp.float32),
        compiler_params=pltpu.CompilerParams(
            dimension_semantics=("parallel",),
        ),
    )(adj, support)

    return out


# speedup vs baseline: 1.0316x; 1.0316x over previous
import jax
import jax.numpy as jnp
from jax.experimental import pallas as pl
from jax.experimental.pallas import tpu as pltpu

_BI = 200
_CS = 1000


def _fused_body(f_ref, w_ref, adj_ref, o_ref, s_ref):
    t = pl.program_id(0)
    n_rows = f_ref.shape[0]

    @pl.when(t == 0)
    def _support():
        def body(j, carry):
            blk = f_ref[pl.ds(j * _CS, _CS), :].astype(jnp.bfloat16)
            s_ref[pl.ds(j * _CS, _CS), :] = jnp.dot(
                blk, w_ref[...], preferred_element_type=jnp.float32
            ).astype(jnp.bfloat16)
            return carry

        jax.lax.fori_loop(0, n_rows // _CS, body, 0)

    o_ref[...] = jnp.maximum(
        jnp.dot(
            adj_ref[...].astype(jnp.bfloat16),
            s_ref[...],
            preferred_element_type=jnp.float32,
        ),
        0.0,
    )


def kernel(features, adj, weight):
    n, d_in = features.shape
    d_out = weight.shape[1]

    return pl.pallas_call(
        _fused_body,
        grid=(n // _BI,),
        in_specs=[
            pl.BlockSpec((n, d_in), lambda i: (0, 0)),
            pl.BlockSpec((d_in, d_out), lambda i: (0, 0)),
            pl.BlockSpec((_BI, n), lambda i: (i, 0)),
        ],
        out_specs=pl.BlockSpec((_BI, d_out), lambda i: (i, 0)),
        out_shape=jax.ShapeDtypeStruct((n, d_out), jnp.float32),
        scratch_shapes=[pltpu.VMEM((n, d_out), jnp.bfloat16)],
        compiler_params=pltpu.CompilerParams(
            dimension_semantics=("arbitrary",),
        ),
    )(features, weight.astype(jnp.bfloat16), adj)
